# baseline (speedup 1.0000x reference)
import functools

import jax
import jax.numpy as jnp
import numpy as np
from jax import lax
from jax.experimental import pallas as pl
from jax.experimental.pallas import tpu as pltpu
from jax.experimental.pallas import tpu_sc as plsc

N = 50000
K = 5000
M = 32
M2 = 4
C = 128
NW = 32

_MESH = dict(core_axis_name="c", subcore_axis_name="s", num_cores=2,
             num_subcores=16)


def _make_sc_gather_sum(group, ch_full, span, tail_rows, nb):
    gpc = 128 // group
    rem = span - nb * (span // nb)
    assert span >= nb >= 2

    @functools.partial(
        pl.kernel,
        out_type=jax.ShapeDtypeStruct((ch_full * gpc + tail_rows, 128),
                                      jnp.float32),
        mesh=plsc.VectorSubcoreMesh(**_MESH),
        scratch_types=[
            pltpu.VMEM((span * 128,), jnp.int32),
            pltpu.VMEM((nb, 128, 64), jnp.int32),
            pltpu.VMEM((nb, gpc, 128), jnp.float32),
        ] + [pltpu.SemaphoreType.DMA] * (2 * nb),
        compiler_params=pltpu.CompilerParams(use_tc_tiling_on_sc=False),
    )
    def gsum(table_hbm, idx_hbm, out_hbm, idx_v, buf_v, acc_v, *sems):
        gsem = sems[:nb]
        osem = sems[nb:]
        wid = lax.axis_index("s") * 2 + lax.axis_index("c")
        lo = (wid * (ch_full - span)) // (NW - 1)
        pltpu.sync_copy(idx_hbm.at[pl.ds(lo * 128, span * 128)], idx_v)

        def _gather_half(b, g, h):
            return pltpu.make_async_copy(
                table_hbm.at[idx_v.at[pl.ds(g * 128 + 64 * h, 64)]],
                buf_v.at[b, pl.ds(64 * h, 64)], gsem[b])

        def _gather_start(b, g):
            _gather_half(b, g, 0).start()
            _gather_half(b, g, 1).start()

        def _gather_wait(b, g):
            _gather_half(b, g, 0).wait()
            _gather_half(b, g, 1).wait()

        def _out_write(b, g):
            return pltpu.make_async_copy(
                acc_v.at[b], out_hbm.at[pl.ds((lo + g) * gpc, gpc)], osem[b])

        def _unpack(b, row, j):
            w = buf_v[b, row, pl.ds(16 * j, 16)]
            lo = lax.bitcast_convert_type(w << 16, jnp.float32)
            hi = lax.bitcast_convert_type(w & jnp.int32(-65536), jnp.float32)
            return lo, hi

        def _reduce(b, nrows=gpc):
            def e_body(e, carry):
                for j in range(4):
                    lo, hi = _unpack(b, e * group, j)
                    for m in range(1, group):
                        lo2, hi2 = _unpack(b, e * group + m, j)
                        lo = lo + lo2
                        hi = hi + hi2
                    acc_v[b, e, pl.ds(16 * j, 16)] = lo
                    acc_v[b, e, pl.ds(64 + 16 * j, 16)] = hi
                return carry
            lax.fori_loop(0, nrows, e_body, 0)

        for b in range(nb):
            _gather_start(b, b)

        def body(p, carry):
            for b in range(nb):
                g = p * nb + b
                _gather_wait(b, g)

                @pl.when(g >= nb)
                def _():
                    _out_write(b, g - nb).wait()

                _reduce(b)
                _out_write(b, g).start()

                @pl.when(g + nb < span)
                def _():
                    _gather_start(b, g + nb)
            return carry

        lax.fori_loop(0, span // nb, body, 0)

        for r in range(rem):
            g_last = span - rem + r
            b = g_last % nb
            _gather_wait(b, g_last)
            _out_write(b, g_last - nb).wait()
            _reduce(b)
            _out_write(b, g_last).start()
        for r in range(nb):
            b = (span - nb + r) % nb
            _out_write(b, 0).wait()

        if tail_rows:
            @pl.when(wid == NW - 1)
            def _():
                nidx = tail_rows * group
                pltpu.sync_copy(idx_hbm.at[pl.ds(ch_full * 128, nidx)],
                                idx_v.at[pl.ds(0, nidx)])
                pltpu.async_copy(
                    table_hbm.at[idx_v.at[pl.ds(0, nidx)]],
                    buf_v.at[0, pl.ds(0, nidx)], gsem[0]).wait()
                _reduce(0, nrows=tail_rows)
                pltpu.sync_copy(
                    acc_v.at[0, pl.ds(0, tail_rows)],
                    out_hbm.at[pl.ds(ch_full * gpc, tail_rows)])

    return gsum


_HIGH = jax.lax.Precision.HIGHEST


def _pack_bf16(y):
    u = lax.bitcast_convert_type(y.astype(jnp.bfloat16),
                                 jnp.uint16).astype(jnp.int32)
    return u[:, :64] | (u[:, 64:] << 16)


def _idx_flat_body(x_ref, o_ref):
    xf = x_ref[...].astype(jnp.float32)
    br, cols = x_ref.shape
    g = 128 // cols
    orpb = o_ref.shape[0]
    r_i = lax.broadcasted_iota(jnp.int32, (orpb, br), 0)
    s_i = lax.broadcasted_iota(jnp.int32, (orpb, br), 1)
    parts = []
    for d in range(g):
        sel = (s_i == g * r_i + d).astype(jnp.float32)
        parts.append(lax.dot_general(sel, xf, (((1,), (0,)), ((), ())),
                                     preferred_element_type=jnp.float32,
                                     precision=_HIGH))
    o_ref[...] = jnp.concatenate(parts, axis=1).astype(jnp.int32)


def _make_idx_flatten(rows, cols, block_rows):
    orpb = block_rows * cols // 128
    assert orpb % 8 == 0
    grid = pl.cdiv(rows, block_rows)
    return pl.pallas_call(
        _idx_flat_body,
        grid=(grid,),
        in_specs=[pl.BlockSpec((block_rows, cols), lambda i: (i, 0))],
        out_specs=pl.BlockSpec((orpb, 128), lambda i: (i, 0)),
        out_shape=jax.ShapeDtypeStruct((grid * orpb, 128), jnp.int32),
    )


def _p1_body(x_ref, w1_ref, o_ref):
    o_ref[...] = _pack_bf16(lax.dot_general(
        x_ref[...], w1_ref[...], (((0,), (0,)), ((), ())),
        preferred_element_type=jnp.float32, precision=_HIGH))


def _p2_body(a_ref, cen_ref, w2_ref, w3_ref, b1_ref, b2_ref, ep_ref, o_ref):
    h = jnp.maximum(a_ref[...] + b1_ref[...], 0.0)
    h = jnp.dot(h, w2_ref[...], preferred_element_type=jnp.float32,
                precision=_HIGH) + b2_ref[...]
    he = h + ep_ref[0, 0] * cen_ref[...].T
    o_ref[...] = _pack_bf16(jnp.dot(he, w3_ref[...],
                                    preferred_element_type=jnp.float32,
                                    precision=_HIGH))


def _p3_body(z_ref, w4_ref, b3_ref, b4_ref, o_ref):
    r = jnp.maximum(z_ref[...] + b3_ref[...], 0.0)
    o_ref[...] = lax.dot_general(
        w4_ref[...], r, (((0,), (1,)), ((), ())),
        preferred_element_type=jnp.float32, precision=_HIGH) + b4_ref[...]


def _run_tc(interpret=False):
    bn = 2048
    p1 = pl.pallas_call(
        _p1_body,
        grid=(pl.cdiv(N, bn),),
        in_specs=[pl.BlockSpec((C, bn), lambda i: (0, i)),
                  pl.BlockSpec((C, C), lambda i: (0, 0))],
        out_specs=pl.BlockSpec((bn, C // 2), lambda i: (i, 0)),
        out_shape=jax.ShapeDtypeStruct((N, C // 2), jnp.int32),
        interpret=interpret,
    )
    bk = 512
    full = lambda i: (0, 0)
    p2 = pl.pallas_call(
        _p2_body,
        grid=(pl.cdiv(K, bk),),
        in_specs=[pl.BlockSpec((bk, C), lambda i: (i, 0)),
                  pl.BlockSpec((C, bk), lambda i: (0, i)),
                  pl.BlockSpec((C, C), full),
                  pl.BlockSpec((C, C), full),
                  pl.BlockSpec((1, C), full),
                  pl.BlockSpec((1, C), full),
                  pl.BlockSpec((1, 1), full)],
        out_specs=pl.BlockSpec((bk, C // 2), lambda i: (i, 0)),
        out_shape=jax.ShapeDtypeStruct((K, C // 2), jnp.int32),
        interpret=interpret,
    )
    p3 = pl.pallas_call(
        _p3_body,
        grid=(pl.cdiv(N, bn),),
        in_specs=[pl.BlockSpec((bn, C), lambda i: (i, 0)),
                  pl.BlockSpec((C, C), full),
                  pl.BlockSpec((1, C), full),
                  pl.BlockSpec((C, 1), full)],
        out_specs=pl.BlockSpec((C, bn), lambda i: (0, i)),
        out_shape=jax.ShapeDtypeStruct((C, N), jnp.float32),
        interpret=interpret,
    )
    return p1, p2, p3


def kernel(node_features, hyperedge_matrix, point_hyperedge_index,
           hyperedge_centers, W1, b1, W2, b2, W3, b3, W4, b4, eps):
    x = node_features[0, :, :, 0]
    hm = hyperedge_matrix[0].astype(jnp.int32)
    phi = point_hyperedge_index[0].astype(jnp.int32)
    cen = hyperedge_centers[0]
    ep = (1.0 + eps).astype(jnp.float32).reshape(1, 1)

    p1, p2, p3 = _run_tc()
    hm2 = _make_idx_flatten(K, M, 320)(hm).reshape(-1)
    phi2 = _make_idx_flatten(N, M2, 2048)(phi).reshape(-1)
    sc_a = _make_sc_gather_sum(M, 1250, 40, 0, 7)
    sc_c = _make_sc_gather_sum(M2, 1562, 49, 16, 7)

    y1 = p1(x, W1)
    aggw1 = sc_a(y1, hm2)
    z = p2(aggw1, cen, W2, W3, b1.reshape(1, C),
           b2.reshape(1, C), ep)
    zsum = sc_c(z, phi2)
    out2d = p3(zsum, W4, b3.reshape(1, C),
               b4.reshape(C, 1))
    return out2d[None, :, :, None]


# speedup vs baseline: 1.9505x; 1.9505x over previous
import functools

import jax
import jax.numpy as jnp
from jax import lax
from jax.experimental import pallas as pl
from jax.experimental.pallas import tpu as pltpu
from jax.experimental.pallas import tpu_sc as plsc

N = 50000
K = 5000
M = 32
M2 = 4
C = 128
NW = 32

_MESH = dict(core_axis_name="c", subcore_axis_name="s", num_cores=2,
             num_subcores=16)


def _make_sc_gather_sum(group, ch_full, span, tail_rows, nb):
    gpc = 128 // group
    rem = span - nb * (span // nb)
    assert span >= nb >= 2

    @functools.partial(
        pl.kernel,
        out_type=jax.ShapeDtypeStruct((ch_full * gpc + tail_rows, 128),
                                      jnp.float32),
        mesh=plsc.VectorSubcoreMesh(**_MESH),
        scratch_types=[
            pltpu.VMEM((span * 128,), jnp.int32),
            pltpu.VMEM((nb, 128, 64), jnp.int32),
            pltpu.VMEM((nb, gpc, 128), jnp.float32),
        ] + [pltpu.SemaphoreType.DMA] * (2 * nb),
        compiler_params=pltpu.CompilerParams(use_tc_tiling_on_sc=False),
    )
    def gsum(table_hbm, idx_hbm, out_hbm, idx_v, buf_v, acc_v, *sems):
        gsem = sems[:nb]
        osem = sems[nb:]
        wid = lax.axis_index("s") * 2 + lax.axis_index("c")
        lo = (wid * (ch_full - span)) // (NW - 1)
        pltpu.sync_copy(idx_hbm.at[pl.ds(lo * 128, span * 128)], idx_v)

        def _gather_half(b, g, h):
            return pltpu.make_async_copy(
                table_hbm.at[idx_v.at[pl.ds(g * 128 + 64 * h, 64)]],
                buf_v.at[b, pl.ds(64 * h, 64)], gsem[b])

        def _gather_start(b, g):
            _gather_half(b, g, 0).start()
            _gather_half(b, g, 1).start()

        def _gather_wait(b, g):
            _gather_half(b, g, 0).wait()
            _gather_half(b, g, 1).wait()

        def _out_write(b, g):
            return pltpu.make_async_copy(
                acc_v.at[b], out_hbm.at[pl.ds((lo + g) * gpc, gpc)], osem[b])

        def _unpack(b, row, j):
            w = buf_v[b, row, pl.ds(16 * j, 16)]
            lo = lax.bitcast_convert_type(w << 16, jnp.float32)
            hi = lax.bitcast_convert_type(w & jnp.int32(-65536), jnp.float32)
            return lo, hi

        def _reduce(b, nrows=gpc):
            def e_body(e, carry):
                for j in range(4):
                    lo, hi = _unpack(b, e * group, j)
                    for m in range(1, group):
                        lo2, hi2 = _unpack(b, e * group + m, j)
                        lo = lo + lo2
                        hi = hi + hi2
                    acc_v[b, e, pl.ds(16 * j, 16)] = lo
                    acc_v[b, e, pl.ds(64 + 16 * j, 16)] = hi
                return carry
            lax.fori_loop(0, nrows, e_body, 0)

        for b in range(nb):
            _gather_start(b, b)

        def body(p, carry):
            for b in range(nb):
                g = p * nb + b
                _gather_wait(b, g)

                @pl.when(g >= nb)
                def _():
                    _out_write(b, g - nb).wait()

                _reduce(b)
                _out_write(b, g).start()

                @pl.when(g + nb < span)
                def _():
                    _gather_start(b, g + nb)
            return carry

        lax.fori_loop(0, span // nb, body, 0)

        for r in range(rem):
            g_last = span - rem + r
            b = g_last % nb
            _gather_wait(b, g_last)
            _out_write(b, g_last - nb).wait()
            _reduce(b)
            _out_write(b, g_last).start()
        for r in range(nb):
            b = (span - nb + r) % nb
            _out_write(b, 0).wait()

        if tail_rows:
            @pl.when(wid == NW - 1)
            def _():
                nidx = tail_rows * group
                pltpu.sync_copy(idx_hbm.at[pl.ds(ch_full * 128, nidx)],
                                idx_v.at[pl.ds(0, nidx)])
                pltpu.async_copy(
                    table_hbm.at[idx_v.at[pl.ds(0, nidx)]],
                    buf_v.at[0, pl.ds(0, nidx)], gsem[0]).wait()
                _reduce(0, nrows=tail_rows)
                pltpu.sync_copy(
                    acc_v.at[0, pl.ds(0, tail_rows)],
                    out_hbm.at[pl.ds(ch_full * gpc, tail_rows)])

    return gsum


_HIGH = jax.lax.Precision.HIGHEST


def _pack_bf16(y):
    u = lax.bitcast_convert_type(y.astype(jnp.bfloat16),
                                 jnp.uint16).astype(jnp.int32)
    return u[:, :64] | (u[:, 64:] << 16)


def _p1_body(x_ref, w1_ref, o_ref):
    o_ref[...] = _pack_bf16(lax.dot_general(
        x_ref[...], w1_ref[...], (((0,), (0,)), ((), ())),
        preferred_element_type=jnp.float32, precision=_HIGH))


def _p2_body(a_ref, cen_ref, w2_ref, w3_ref, b1_ref, b2_ref, ep_ref, o_ref):
    h = jnp.maximum(a_ref[...] + b1_ref[...], 0.0)
    h = jnp.dot(h, w2_ref[...], preferred_element_type=jnp.float32,
                precision=_HIGH) + b2_ref[...]
    he = h + ep_ref[0, 0] * cen_ref[...].T
    o_ref[...] = _pack_bf16(jnp.dot(he, w3_ref[...],
                                    preferred_element_type=jnp.float32,
                                    precision=_HIGH))


def _p3_body(z_ref, w4_ref, b3_ref, b4_ref, o_ref):
    r = jnp.maximum(z_ref[...] + b3_ref[...], 0.0)
    o_ref[...] = lax.dot_general(
        w4_ref[...], r, (((0,), (1,)), ((), ())),
        preferred_element_type=jnp.float32, precision=_HIGH) + b4_ref[...]


def _run_tc(interpret=False):
    bn = 2048
    p1 = pl.pallas_call(
        _p1_body,
        grid=(pl.cdiv(N, bn),),
        in_specs=[pl.BlockSpec((C, bn), lambda i: (0, i)),
                  pl.BlockSpec((C, C), lambda i: (0, 0))],
        out_specs=pl.BlockSpec((bn, C // 2), lambda i: (i, 0)),
        out_shape=jax.ShapeDtypeStruct((N, C // 2), jnp.int32),
        interpret=interpret,
    )
    bk = 512
    full = lambda i: (0, 0)
    p2 = pl.pallas_call(
        _p2_body,
        grid=(pl.cdiv(K, bk),),
        in_specs=[pl.BlockSpec((bk, C), lambda i: (i, 0)),
                  pl.BlockSpec((C, bk), lambda i: (0, i)),
                  pl.BlockSpec((C, C), full),
                  pl.BlockSpec((C, C), full),
                  pl.BlockSpec((1, C), full),
                  pl.BlockSpec((1, C), full),
                  pl.BlockSpec((1, 1), full)],
        out_specs=pl.BlockSpec((bk, C // 2), lambda i: (i, 0)),
        out_shape=jax.ShapeDtypeStruct((K, C // 2), jnp.int32),
        interpret=interpret,
    )
    p3 = pl.pallas_call(
        _p3_body,
        grid=(pl.cdiv(N, bn),),
        in_specs=[pl.BlockSpec((bn, C), lambda i: (i, 0)),
                  pl.BlockSpec((C, C), full),
                  pl.BlockSpec((1, C), full),
                  pl.BlockSpec((C, 1), full)],
        out_specs=pl.BlockSpec((C, bn), lambda i: (0, i)),
        out_shape=jax.ShapeDtypeStruct((C, N), jnp.float32),
        interpret=interpret,
    )
    return p1, p2, p3


def kernel(node_features, hyperedge_matrix, point_hyperedge_index,
           hyperedge_centers, W1, b1, W2, b2, W3, b3, W4, b4, eps):
    x = node_features[0, :, :, 0]
    hm = hyperedge_matrix[0].astype(jnp.int32)
    phi = point_hyperedge_index[0].astype(jnp.int32)
    cen = hyperedge_centers[0]
    ep = (1.0 + eps).astype(jnp.float32).reshape(1, 1)

    p1, p2, p3 = _run_tc()
    hm2 = hm.reshape(-1)
    phi2 = phi.reshape(-1)
    sc_a = _make_sc_gather_sum(M, 1250, 40, 0, 7)
    sc_c = _make_sc_gather_sum(M2, 1562, 49, 16, 7)

    y1 = p1(x, W1)
    aggw1 = sc_a(y1, hm2)
    z = p2(aggw1, cen, W2, W3, b1.reshape(1, C),
           b2.reshape(1, C), ep)
    zsum = sc_c(z, phi2)
    out2d = p3(zsum, W4, b3.reshape(1, C),
               b4.reshape(C, 1))
    return out2d[None, :, :, None]
